# baseline (device time: 21497 ns/iter reference)
import jax
import jax.numpy as jnp
from jax import lax
from jax.experimental import pallas as pl
from jax.experimental.pallas import tpu as pltpu

_DeviceIdType = getattr(pl, "DeviceIdType", None) or pltpu.DeviceIdType
_sem_signal = getattr(pl, "semaphore_signal", None) or pltpu.semaphore_signal
_sem_wait = getattr(pl, "semaphore_wait", None) or pltpu.semaphore_wait
_CompilerParams = getattr(pltpu, "CompilerParams", None) or getattr(
    pltpu, "TPUCompilerParams"
)


def kernel(x, dest):
    n_per, d = x.shape
    dest2 = dest.reshape(1, n_per)

    def body(x_ref, d_ref, out_ref, comm_x, comm_d, send_sems, recv_sems):
        my_x = lax.axis_index("x")
        my_y = lax.axis_index("y")
        my_z = lax.axis_index("z")
        peer = (1 - my_x, my_y, my_z)

        barrier_sem = pltpu.get_barrier_semaphore()
        _sem_signal(
            barrier_sem, inc=1, device_id=peer, device_id_type=_DeviceIdType.MESH
        )
        _sem_wait(barrier_sem, 1)

        comm_x[0, :, :] = x_ref[:, :].astype(jnp.bfloat16)
        comm_d[0, :] = d_ref[0, :]

        rdma_x = pltpu.make_async_remote_copy(
            src_ref=comm_x.at[0],
            dst_ref=comm_x.at[1],
            send_sem=send_sems.at[0],
            recv_sem=recv_sems.at[0],
            device_id=peer,
            device_id_type=_DeviceIdType.MESH,
        )
        rdma_d = pltpu.make_async_remote_copy(
            src_ref=comm_d.at[0],
            dst_ref=comm_d.at[1],
            send_sem=send_sems.at[1],
            recv_sem=recv_sems.at[1],
            device_id=peer,
            device_id_type=_DeviceIdType.MESH,
        )
        rdma_x.start()
        rdma_d.start()
        rdma_d.wait()

        d_all = comm_d[:, :]
        maskf = (d_all == my_x).astype(jnp.float32)
        ii = lax.broadcasted_iota(jnp.int32, (n_per, n_per), 0)
        jj = lax.broadcasted_iota(jnp.int32, (n_per, n_per), 1)
        tri = (ii <= jj).astype(jnp.float32)
        cum = lax.dot_general(
            maskf, tri, (((1,), (0,)), ((), ())),
            preferred_element_type=jnp.float32,
        )
        tot = lax.slice(cum, (0, n_per - 1), (2, n_per))
        t_mine = lax.slice(tot, (0, 0), (1, 1))
        t_peer = lax.slice(tot, (1, 0), (2, 1))
        off0 = jnp.where(my_x == 0, 0.0, t_peer)
        off1 = jnp.where(my_x == 0, t_mine, 0.0)
        m0 = lax.slice(maskf, (0, 0), (1, n_per))
        m1 = lax.slice(maskf, (1, 0), (2, n_per))
        c0 = lax.slice(cum, (0, 0), (1, n_per))
        c1 = lax.slice(cum, (1, 0), (2, n_per))
        pos0 = jnp.where(m0 > 0, c0 - 1.0 + off0, -1.0)
        pos1 = jnp.where(m1 > 0, c1 - 1.0 + off1, -1.0)
        rowi = lax.broadcasted_iota(jnp.int32, (n_per, n_per), 0).astype(
            jnp.float32
        )
        p0 = (pos0 == rowi).astype(jnp.bfloat16)
        p1 = (pos1 == rowi).astype(jnp.bfloat16)
        acc = lax.dot_general(
            p0, comm_x[0], (((1,), (0,)), ((), ())),
            preferred_element_type=jnp.float32,
        )
        rdma_x.wait()
        acc = acc + lax.dot_general(
            p1, comm_x[1], (((1,), (0,)), ((), ())),
            preferred_element_type=jnp.float32,
        )
        out_ref[:, :] = acc

    return pl.pallas_call(
        body,
        out_shape=jax.ShapeDtypeStruct((n_per, d), jnp.float32),
        in_specs=[
            pl.BlockSpec(memory_space=pltpu.VMEM),
            pl.BlockSpec(memory_space=pltpu.VMEM),
        ],
        out_specs=pl.BlockSpec(memory_space=pltpu.VMEM),
        scratch_shapes=[
            pltpu.VMEM((2, n_per, d), jnp.bfloat16),
            pltpu.VMEM((2, n_per), jnp.int32),
            pltpu.SemaphoreType.DMA((2,)),
            pltpu.SemaphoreType.DMA((2,)),
        ],
        compiler_params=_CompilerParams(collective_id=0),
    )(x, dest2)


# device time: 15844 ns/iter; 1.3568x vs baseline; 1.3568x over previous
import jax
import jax.numpy as jnp
from jax import lax
from jax.experimental import pallas as pl
from jax.experimental.pallas import tpu as pltpu

_DeviceIdType = getattr(pl, "DeviceIdType", None) or pltpu.DeviceIdType
_sem_signal = getattr(pl, "semaphore_signal", None) or pltpu.semaphore_signal
_sem_wait = getattr(pl, "semaphore_wait", None) or pltpu.semaphore_wait
_CompilerParams = getattr(pltpu, "CompilerParams", None) or getattr(
    pltpu, "TPUCompilerParams"
)

_CHUNK = 128


def kernel(x, dest):
    n_per, d = x.shape
    n_chunks = n_per // _CHUNK
    dest2 = dest.reshape(1, n_per)

    def body(x_ref, d_ref, out_ref, y_send, recv_buf, comm_d, send_sems, recv_sems):
        my_x = lax.axis_index("x")
        my_y = lax.axis_index("y")
        my_z = lax.axis_index("z")
        peer = (1 - my_x, my_y, my_z)
        f32 = jnp.float32

        barrier_sem = pltpu.get_barrier_semaphore()
        _sem_signal(
            barrier_sem, inc=1, device_id=peer, device_id_type=_DeviceIdType.MESH
        )
        _sem_wait(barrier_sem, 1)

        comm_d[0, :] = d_ref[0, :]
        rdma_d = pltpu.make_async_remote_copy(
            src_ref=comm_d.at[0],
            dst_ref=comm_d.at[1],
            send_sem=send_sems.at[n_chunks],
            recv_sem=recv_sems.at[n_chunks],
            device_id=peer,
            device_id_type=_DeviceIdType.MESH,
        )
        rdma_d.start()

        d_loc = d_ref[:, :]
        keep = d_loc == my_x
        keepf = keep.astype(f32)
        ii = lax.broadcasted_iota(jnp.int32, (n_per, n_per), 0)
        jj = lax.broadcasted_iota(jnp.int32, (n_per, n_per), 1)
        tri = (ii <= jj).astype(f32)
        iif = ii.astype(f32)
        cumk = lax.dot_general(
            keepf, tri, (((1,), (0,)), ((), ())), preferred_element_type=f32
        )
        col = lax.broadcasted_iota(jnp.int32, (1, n_per), 1).astype(f32)
        cums = col + 1.0 - cumk

        pos_send = jnp.where(keep, -1.0, cums - 1.0)
        p_send = (pos_send == iif).astype(jnp.bfloat16)
        xb = x_ref[:, :].astype(jnp.bfloat16)
        y_send[:, :] = lax.dot_general(
            p_send, xb, (((1,), (0,)), ((), ())), preferred_element_type=f32
        ).astype(jnp.bfloat16)

        k_keep = jnp.sum(keep.astype(jnp.int32))
        n_send = n_per - k_keep
        send_descs = []
        for c in range(n_chunks):
            active = c * _CHUNK < n_send
            rdma_c = pltpu.make_async_remote_copy(
                src_ref=y_send.at[pl.ds(c * _CHUNK, _CHUNK)],
                dst_ref=recv_buf.at[pl.ds(c * _CHUNK, _CHUNK)],
                send_sem=send_sems.at[c],
                recv_sem=recv_sems.at[c],
                device_id=peer,
                device_id_type=_DeviceIdType.MESH,
            )
            send_descs.append((active, rdma_c))

            @pl.when(active)
            def _(rdma_c=rdma_c):
                rdma_c.start()

        rdma_d.wait()
        peer_d = comm_d[1:2, :]
        n_recv = jnp.sum((peer_d == my_x).astype(jnp.int32))
        off_keep = jnp.where(my_x == 0, 0, n_recv).astype(f32)
        off_recv = jnp.where(my_x == 0, n_per - n_recv, 0).astype(f32)
        n_recv_f = n_recv.astype(f32)

        pos_keep = jnp.where(keep, off_keep + cumk - 1.0, -1.0)
        p_keep = (pos_keep == iif).astype(jnp.bfloat16)
        acc = lax.dot_general(
            p_keep, xb, (((1,), (0,)), ((), ())), preferred_element_type=f32
        )

        tt = lax.broadcasted_iota(jnp.int32, (1, _CHUNK), 1).astype(f32)
        rowc = lax.broadcasted_iota(jnp.int32, (n_per, _CHUNK), 0).astype(f32)
        for c in range(n_chunks):

            @pl.when(c * _CHUNK < n_recv)
            def _(c=c):
                recv_c = pltpu.make_async_remote_copy(
                    src_ref=y_send.at[pl.ds(c * _CHUNK, _CHUNK)],
                    dst_ref=recv_buf.at[pl.ds(c * _CHUNK, _CHUNK)],
                    send_sem=send_sems.at[c],
                    recv_sem=recv_sems.at[c],
                    device_id=peer,
                    device_id_type=_DeviceIdType.MESH,
                )
                recv_c.wait_recv()

            g = c * _CHUNK + tt
            pos_c = jnp.where(g < n_recv_f, off_recv + g, -1.0)
            p_c = (pos_c == rowc).astype(jnp.bfloat16)
            acc = acc + lax.dot_general(
                p_c,
                recv_buf[pl.ds(c * _CHUNK, _CHUNK), :],
                (((1,), (0,)), ((), ())),
                preferred_element_type=f32,
            )

        out_ref[:, :] = acc.astype(jnp.bfloat16)

        for active, rdma_c in send_descs:

            @pl.when(active)
            def _(rdma_c=rdma_c):
                rdma_c.wait_send()

    return pl.pallas_call(
        body,
        out_shape=jax.ShapeDtypeStruct((n_per, d), jnp.bfloat16),
        in_specs=[
            pl.BlockSpec(memory_space=pltpu.VMEM),
            pl.BlockSpec(memory_space=pltpu.VMEM),
        ],
        out_specs=pl.BlockSpec(memory_space=pltpu.VMEM),
        scratch_shapes=[
            pltpu.VMEM((n_per, d), jnp.bfloat16),
            pltpu.VMEM((n_per, d), jnp.bfloat16),
            pltpu.VMEM((2, n_per), jnp.int32),
            pltpu.SemaphoreType.DMA((n_per // _CHUNK + 1,)),
            pltpu.SemaphoreType.DMA((n_per // _CHUNK + 1,)),
        ],
        compiler_params=_CompilerParams(collective_id=0),
    )(x, dest2)


# device time: 15344 ns/iter; 1.4010x vs baseline; 1.0326x over previous
import jax
import jax.numpy as jnp
from jax import lax
from jax.experimental import pallas as pl
from jax.experimental.pallas import tpu as pltpu

_DeviceIdType = getattr(pl, "DeviceIdType", None) or pltpu.DeviceIdType
_sem_signal = getattr(pl, "semaphore_signal", None) or pltpu.semaphore_signal
_sem_wait = getattr(pl, "semaphore_wait", None) or pltpu.semaphore_wait
_CompilerParams = getattr(pltpu, "CompilerParams", None) or getattr(
    pltpu, "TPUCompilerParams"
)

_CHUNK = 128


def kernel(x, dest):
    n_per, d = x.shape
    n_chunks = n_per // _CHUNK
    dest2 = dest.reshape(1, n_per)

    def body(x_ref, d_ref, out_ref, y_send, recv_buf, comm_d, send_sems, recv_sems):
        my_x = lax.axis_index("x")
        my_y = lax.axis_index("y")
        my_z = lax.axis_index("z")
        peer = (1 - my_x, my_y, my_z)
        f32 = jnp.float32

        barrier_sem = pltpu.get_barrier_semaphore()
        _sem_signal(
            barrier_sem, inc=1, device_id=peer, device_id_type=_DeviceIdType.MESH
        )
        _sem_wait(barrier_sem, 1)

        comm_d[0, :] = d_ref[0, :]
        rdma_d = pltpu.make_async_remote_copy(
            src_ref=comm_d.at[0],
            dst_ref=comm_d.at[1],
            send_sem=send_sems.at[n_chunks],
            recv_sem=recv_sems.at[n_chunks],
            device_id=peer,
            device_id_type=_DeviceIdType.MESH,
        )
        rdma_d.start()

        d_loc = d_ref[:, :]
        keep = d_loc == my_x
        s = keep.astype(jnp.int32)
        k = 1
        while k < n_per:
            s = s + jnp.concatenate(
                [jnp.zeros((1, k), jnp.int32), s[:, : n_per - k]], axis=1
            )
            k *= 2
        cumk = s.astype(f32)
        col = lax.broadcasted_iota(jnp.int32, (1, n_per), 1).astype(f32)
        cums = col + 1.0 - cumk

        k_keep = jnp.sum(keep.astype(jnp.int32))
        n_send = n_per - k_keep
        pos_send = jnp.where(keep, -1.0, cums - 1.0)
        xb = x_ref[:, :].astype(jnp.bfloat16)

        chunk_rows = lax.broadcasted_iota(jnp.int32, (_CHUNK, n_per), 0).astype(f32)
        send_descs = []
        for c in range(n_chunks):
            active = c * _CHUNK < n_send
            rdma_c = pltpu.make_async_remote_copy(
                src_ref=y_send.at[pl.ds(c * _CHUNK, _CHUNK)],
                dst_ref=recv_buf.at[pl.ds(c * _CHUNK, _CHUNK)],
                send_sem=send_sems.at[c],
                recv_sem=recv_sems.at[c],
                device_id=peer,
                device_id_type=_DeviceIdType.MESH,
            )
            send_descs.append((active, rdma_c))

            @pl.when(active)
            def _(rdma_c=rdma_c, c=c):
                p_c = (pos_send == (chunk_rows + float(c * _CHUNK))).astype(
                    jnp.bfloat16
                )
                y_send[pl.ds(c * _CHUNK, _CHUNK), :] = lax.dot_general(
                    p_c, xb, (((1,), (0,)), ((), ())), preferred_element_type=f32
                ).astype(jnp.bfloat16)
                rdma_c.start()

        rdma_d.wait()
        peer_d = comm_d[1:2, :]
        n_recv = jnp.sum((peer_d == my_x).astype(jnp.int32))
        off_keep = jnp.where(my_x == 0, 0, n_recv).astype(f32)
        off_recv = jnp.where(my_x == 0, n_per - n_recv, 0).astype(f32)
        n_recv_f = n_recv.astype(f32)

        iif = lax.broadcasted_iota(jnp.int32, (n_per, n_per), 0).astype(f32)
        pos_keep = jnp.where(keep, off_keep + cumk - 1.0, -1.0)
        p_keep = (pos_keep == iif).astype(jnp.bfloat16)
        acc = lax.dot_general(
            p_keep, xb, (((1,), (0,)), ((), ())), preferred_element_type=f32
        )

        tt = lax.broadcasted_iota(jnp.int32, (1, _CHUNK), 1).astype(f32)
        rowc = lax.broadcasted_iota(jnp.int32, (n_per, _CHUNK), 0).astype(f32)
        gcol = lax.broadcasted_iota(jnp.int32, (_CHUNK, 1), 0).astype(f32)
        for c in range(n_chunks):

            @pl.when(c * _CHUNK < n_recv)
            def _(c=c):
                recv_c = pltpu.make_async_remote_copy(
                    src_ref=y_send.at[pl.ds(c * _CHUNK, _CHUNK)],
                    dst_ref=recv_buf.at[pl.ds(c * _CHUNK, _CHUNK)],
                    send_sem=send_sems.at[c],
                    recv_sem=recv_sems.at[c],
                    device_id=peer,
                    device_id_type=_DeviceIdType.MESH,
                )
                recv_c.wait_recv()

            g = c * _CHUNK + tt
            pos_c = jnp.where(g < n_recv_f, off_recv + g, -1.0)
            p_c = (pos_c == rowc).astype(jnp.bfloat16)
            raw = recv_buf[pl.ds(c * _CHUNK, _CHUNK), :]
            clean = jnp.where(
                (float(c * _CHUNK) + gcol) < n_recv_f,
                raw,
                jnp.zeros_like(raw),
            )
            acc = acc + lax.dot_general(
                p_c,
                clean,
                (((1,), (0,)), ((), ())),
                preferred_element_type=f32,
            )

        out_ref[:, :] = acc.astype(jnp.bfloat16)

        for active, rdma_c in send_descs:

            @pl.when(active)
            def _(rdma_c=rdma_c):
                rdma_c.wait_send()

    return pl.pallas_call(
        body,
        out_shape=jax.ShapeDtypeStruct((n_per, d), jnp.bfloat16),
        in_specs=[
            pl.BlockSpec(memory_space=pltpu.VMEM),
            pl.BlockSpec(memory_space=pltpu.VMEM),
        ],
        out_specs=pl.BlockSpec(memory_space=pltpu.VMEM),
        scratch_shapes=[
            pltpu.VMEM((n_per, d), jnp.bfloat16),
            pltpu.VMEM((n_per, d), jnp.bfloat16),
            pltpu.VMEM((2, n_per), jnp.int32),
            pltpu.SemaphoreType.DMA((n_per // _CHUNK + 1,)),
            pltpu.SemaphoreType.DMA((n_per // _CHUNK + 1,)),
        ],
        compiler_params=_CompilerParams(collective_id=0),
    )(x, dest2)
